# baseline (speedup 1.0000x reference)
import functools
import math

import jax
import jax.numpy as jnp
from jax import lax
from jax.experimental import pallas as pl
from jax.experimental.pallas import tpu as pltpu
from jax.experimental.pallas import tpu_sc as plsc

_D = 128
_NSEEDS = 500
_NPAD = 512
_K = 4
_BBLK = 2048
_NEG = -1e30

_NW = 32
_BPW = 16384 // _NW
_HALF = _BPW // 2


def _tc_body(x_ref, seeds_ref, wq_ref, attn_ref, idx_ref):
    x = x_ref[...]
    seeds = seeds_ref[...]
    wq = wq_ref[...]
    q = jax.lax.dot_general(x, wq, (((1,), (1,)), ((), ())),
                            preferred_element_type=jnp.float32)
    s = jax.lax.dot_general(q, seeds, (((1,), (1,)), ((), ())),
                            preferred_element_type=jnp.float32)
    s = s * (1.0 / math.sqrt(_D))
    cols = jax.lax.broadcasted_iota(jnp.int32, (_BBLK, _NPAD), 1)
    s = jnp.where(cols < _NSEEDS, s, _NEG)

    v1 = jnp.max(s, axis=1, keepdims=True)
    s1 = jnp.where(s >= v1, _NEG, s)
    v2 = jnp.max(s1, axis=1, keepdims=True)
    s2 = jnp.where(s1 >= v2, _NEG, s1)
    v3 = jnp.max(s2, axis=1, keepdims=True)
    s3 = jnp.where(s2 >= v3, _NEG, s2)
    v4 = jnp.max(s3, axis=1, keepdims=True)

    e2 = jnp.exp(v2 - v1)
    e3 = jnp.exp(v3 - v1)
    e4 = jnp.exp(v4 - v1)
    rz = 1.0 / (1.0 + e2 + e3 + e4)
    attn_ref[...] = jnp.concatenate([jnp.ones_like(v1), e2, e3, e4],
                                    axis=1) * rz

    ix = [jnp.min(jnp.where(s == v, cols, _NPAD), axis=1, keepdims=True)
          for v in (v1, v2, v3, v4)]
    idx_ref[...] = jnp.minimum(jnp.concatenate(ix, axis=1), _NSEEDS - 1)


def _tc_stage(x, seeds_p, Wq):
    batch = x.shape[0]
    grid = (batch // _BBLK,)
    return pl.pallas_call(
        _tc_body,
        grid=grid,
        in_specs=[
            pl.BlockSpec((_BBLK, _D), lambda i: (i, 0)),
            pl.BlockSpec((_NPAD, _D), lambda i: (0, 0)),
            pl.BlockSpec((_D, _D), lambda i: (0, 0)),
        ],
        out_specs=[
            pl.BlockSpec((_BBLK, _K), lambda i: (i, 0)),
            pl.BlockSpec((_BBLK, _K), lambda i: (i, 0)),
        ],
        out_shape=[
            jax.ShapeDtypeStruct((batch, _K), jnp.float32),
            jax.ShapeDtypeStruct((batch, _K), jnp.int32),
        ],
    )(x, seeds_p, Wq)


def _sc_body(seeds_hbm, idx_hbm, attn_hbm, field_hbm,
             seeds_v, idx_v, attn_v, out_v):
    wid = lax.axis_index("s") * 2 + lax.axis_index("c")
    base = wid * _BPW
    pltpu.sync_copy(seeds_hbm, seeds_v)
    pltpu.sync_copy(idx_hbm.at[:, pl.ds(base, _BPW)], idx_v)
    pltpu.sync_copy(attn_hbm.at[:, pl.ds(base, _BPW)], attn_v)
    lane = lax.iota(jnp.int32, 16)

    for half in range(2):
        for g in range(_HALF // 16):
            rowb = g * 16
            srcb = half * _HALF + rowb
            ivs = [idx_v[k, srcb:srcb + 16] for k in range(_K)]
            avs = [attn_v[k, srcb:srcb + 16] for k in range(_K)]
            rowv = lane + rowb

            def col_body(c, carry2, ivs=ivs, avs=avs, rowv=rowv):
                c4 = c * 4
                for u in range(4):
                    csplat = jnp.full((16,), c4 + u, jnp.int32)
                    acc = jnp.zeros((16,), jnp.float32)
                    for k in range(_K):
                        sv = plsc.load_gather(seeds_v, [ivs[k], csplat])
                        acc = acc + avs[k] * sv
                    plsc.store_scatter(out_v, [rowv, csplat], acc)
                return carry2

            lax.fori_loop(0, _D // 4, col_body, 0)

        pltpu.sync_copy(
            out_v, field_hbm.at[pl.ds(base + half * _HALF, _HALF)])


def _sc_stage(seeds, idx, attn):
    batch = idx.shape[0]
    mesh = plsc.VectorSubcoreMesh(core_axis_name="c", subcore_axis_name="s")
    run = functools.partial(
        pl.kernel,
        mesh=mesh,
        compiler_params=pltpu.CompilerParams(needs_layout_passes=False),
        out_type=jax.ShapeDtypeStruct((batch, _D), jnp.float32),
        scratch_types=[
            pltpu.VMEM((_NSEEDS, _D), jnp.float32),
            pltpu.VMEM((_K, _BPW), jnp.int32),
            pltpu.VMEM((_K, _BPW), jnp.float32),
            pltpu.VMEM((_HALF, _D), jnp.float32),
        ],
    )(_sc_body)
    return run(seeds, idx.T.copy(), attn.T.copy())


def kernel(x, seeds, Wq):
    seeds_p = jnp.zeros((_NPAD, _D), jnp.float32).at[:_NSEEDS].set(seeds)
    attn, idx = _tc_stage(x, seeds_p, Wq)
    field = _sc_stage(seeds, idx, attn)
    return (field, attn)


# speedup vs baseline: 1.0213x; 1.0213x over previous
import functools
import math

import jax
import jax.numpy as jnp
from jax import lax
from jax.experimental import pallas as pl
from jax.experimental.pallas import tpu as pltpu
from jax.experimental.pallas import tpu_sc as plsc

_D = 128
_NSEEDS = 500
_NPAD = 512
_K = 4
_BBLK = 2048
_NEG = -1e30

_NW = 32
_BPW = 16384 // _NW
_HALF = _BPW // 2


def _tc_body(x_ref, seeds_ref, wq_ref, attn_ref, idx_ref):
    x = x_ref[...]
    seeds = seeds_ref[...]
    wq = wq_ref[...]
    q = jax.lax.dot_general(x, wq, (((1,), (1,)), ((), ())),
                            preferred_element_type=jnp.float32)
    s = jax.lax.dot_general(q, seeds, (((1,), (1,)), ((), ())),
                            preferred_element_type=jnp.float32)
    s = s * (1.0 / math.sqrt(_D))
    cols = jax.lax.broadcasted_iota(jnp.int32, (_BBLK, _NPAD), 1)
    s = jnp.where(cols < _NSEEDS, s, _NEG)

    v1 = jnp.max(s, axis=1, keepdims=True)
    s1 = jnp.where(s >= v1, _NEG, s)
    v2 = jnp.max(s1, axis=1, keepdims=True)
    s2 = jnp.where(s1 >= v2, _NEG, s1)
    v3 = jnp.max(s2, axis=1, keepdims=True)
    s3 = jnp.where(s2 >= v3, _NEG, s2)
    v4 = jnp.max(s3, axis=1, keepdims=True)

    e2 = jnp.exp(v2 - v1)
    e3 = jnp.exp(v3 - v1)
    e4 = jnp.exp(v4 - v1)
    rz = 1.0 / (1.0 + e2 + e3 + e4)
    attn_ref[...] = jnp.concatenate([jnp.ones_like(v1), e2, e3, e4],
                                    axis=1) * rz

    ix = [jnp.min(jnp.where(s == v, cols, _NPAD), axis=1, keepdims=True)
          for v in (v1, v2, v3, v4)]
    idx_ref[...] = jnp.minimum(jnp.concatenate(ix, axis=1), _NSEEDS - 1)


def _tc_stage(x, seeds_p, Wq):
    batch = x.shape[0]
    grid = (batch // _BBLK,)
    return pl.pallas_call(
        _tc_body,
        grid=grid,
        in_specs=[
            pl.BlockSpec((_BBLK, _D), lambda i: (i, 0)),
            pl.BlockSpec((_NPAD, _D), lambda i: (0, 0)),
            pl.BlockSpec((_D, _D), lambda i: (0, 0)),
        ],
        out_specs=[
            pl.BlockSpec((_BBLK, _K), lambda i: (i, 0)),
            pl.BlockSpec((_BBLK, _K), lambda i: (i, 0)),
        ],
        out_shape=[
            jax.ShapeDtypeStruct((batch, _K), jnp.float32),
            jax.ShapeDtypeStruct((batch, _K), jnp.int32),
        ],
    )(x, seeds_p, Wq)


def _sc_body(seeds_hbm, idx_hbm, attn_hbm, field_hbm,
             seeds_v, idx_v, attn_v, out_v):
    wid = lax.axis_index("s") * 2 + lax.axis_index("c")
    base = wid * _BPW
    pltpu.sync_copy(seeds_hbm, seeds_v)
    pltpu.sync_copy(idx_hbm.at[:, pl.ds(base, _BPW)], idx_v)
    pltpu.sync_copy(attn_hbm.at[:, pl.ds(base, _BPW)], attn_v)
    lane = lax.iota(jnp.int32, 16)

    for half in range(2):
        for g in range(_HALF // 16):
            rowb = g * 16
            srcb = half * _HALF + rowb
            ivs = [idx_v[k, srcb:srcb + 16] for k in range(_K)]
            avs = [attn_v[k, srcb:srcb + 16] for k in range(_K)]
            rowv = lane + rowb

            def col_body(c, carry2, ivs=ivs, avs=avs, rowv=rowv):
                csplat = jnp.full((16,), c, jnp.int32)
                acc = jnp.zeros((16,), jnp.float32)
                for k in range(_K):
                    sv = plsc.load_gather(seeds_v, [ivs[k], csplat])
                    acc = acc + avs[k] * sv
                plsc.store_scatter(out_v, [rowv, csplat], acc)
                return carry2

            lax.fori_loop(0, _D, col_body, 0)

        pltpu.sync_copy(
            out_v, field_hbm.at[pl.ds(base + half * _HALF, _HALF)])


def _sc_stage(seeds, idx, attn):
    batch = idx.shape[0]
    mesh = plsc.VectorSubcoreMesh(core_axis_name="c", subcore_axis_name="s")
    run = functools.partial(
        pl.kernel,
        mesh=mesh,
        compiler_params=pltpu.CompilerParams(needs_layout_passes=False),
        out_type=jax.ShapeDtypeStruct((batch, _D), jnp.float32),
        scratch_types=[
            pltpu.VMEM((_NSEEDS, _D), jnp.float32),
            pltpu.VMEM((_K, _BPW), jnp.int32),
            pltpu.VMEM((_K, _BPW), jnp.float32),
            pltpu.VMEM((_HALF, _D), jnp.float32),
        ],
    )(_sc_body)
    return run(seeds, idx.T.copy(), attn.T.copy())


def kernel(x, seeds, Wq):
    seeds_p = jnp.zeros((_NPAD, _D), jnp.float32).at[:_NSEEDS].set(seeds)
    attn, idx = _tc_stage(x, seeds_p, Wq)
    field = _sc_stage(seeds, idx, attn)
    return (field, attn)
